# baseline (speedup 1.0000x reference)
import functools

import jax
import jax.numpy as jnp
from jax.experimental import pallas as pl
from jax.experimental.pallas import tpu as pltpu

_N = 4096
_D = 64
_OUT = 64
_BM = 256


def _scn_body(L_ref, x_ref, w1t_ref, w2t_ref, b_ref, o_ref, z_ref):
    i = pl.program_id(0)

    @pl.when(i == 0)
    def _():
        z_ref[...] = jnp.dot(
            x_ref[...], w2t_ref[...], preferred_element_type=jnp.float32
        )

    x_blk = x_ref[pl.ds(i * _BM, _BM), :]
    acc = jnp.dot(L_ref[...], z_ref[...], preferred_element_type=jnp.float32)
    o_ref[...] = (
        acc
        + jnp.dot(x_blk, w1t_ref[...], preferred_element_type=jnp.float32)
        + b_ref[...]
    )


@jax.jit
def kernel(L, x, W, b):
    n, d = x.shape
    out = W.shape[0]
    w1t = W[:, :d].T
    w2t = W[:, d:].T
    b2 = b.reshape(1, out)

    grid = (n // _BM,)
    return pl.pallas_call(
        _scn_body,
        grid=grid,
        in_specs=[
            pl.BlockSpec((_BM, n), lambda i: (i, 0)),
            pl.BlockSpec((n, d), lambda i: (0, 0)),
            pl.BlockSpec((d, out), lambda i: (0, 0)),
            pl.BlockSpec((d, out), lambda i: (0, 0)),
            pl.BlockSpec((1, out), lambda i: (0, 0)),
        ],
        out_specs=pl.BlockSpec((_BM, out), lambda i: (i, 0)),
        out_shape=jax.ShapeDtypeStruct((n, out), jnp.float32),
        scratch_shapes=[pltpu.VMEM((n, out), jnp.float32)],
    )(L, x, w1t, w2t, b2)
